# baseline (speedup 1.0000x reference)
import functools

import jax
import jax.numpy as jnp
from jax.experimental import pallas as pl
from jax.experimental.pallas import tpu as pltpu

_NUM_CODES = 512
_LATENT = 64
_NEG_LOG2E = -1.4426950408889634


_CPT = 2


def _vq_tile(x_ref, lhs_ref, base_ref, soft_ref, hard_ref, idx_ref):
    for i in range(_CPT):
        x = x_ref[i, 0, :]
        w = lhs_ref[i][0, :].reshape(-1, 1)
        t = w - x[None, :]
        d = jnp.abs(t)
        dmin = jnp.min(d, axis=0, keepdims=True)
        e = jnp.exp2(d * jnp.float32(_NEG_LOG2E))
        nd = jax.lax.dot_general(
            lhs_ref[i], e, (((1,), (0,)), ((), ())),
            preferred_element_type=jnp.float32,
        )
        soft_ref[i, 0, :] = nd[0, :] / nd[1, :]
        sbit = jax.lax.shift_right_logical(
            jax.lax.bitcast_convert_type(t, jnp.uint32), jnp.uint32(31)
        )
        key = jax.lax.bitcast_convert_type(base_ref[0] | sbit, jnp.float32)
        packed = jax.lax.bitcast_convert_type(
            jnp.min(jnp.where(d == dmin, key, jnp.float32(2.0)), axis=0),
            jnp.uint32,
        )
        idx_ref[i, 0, :] = (
            (packed >> jnp.uint32(1)) & jnp.uint32(0x3FF)
        ).astype(jnp.int32)
        sign = 1.0 - 2.0 * (packed & jnp.uint32(1)).astype(jnp.float32)
        hard_ref[i, 0, :] = x + sign * dmin[0, :]


@functools.partial(jax.jit, static_argnames=("interpret",))
def _run(z, codes, interpret=False):
    B, C, H, W = z.shape
    K = codes.shape[1]
    S = B * H * W
    SBLK = 2304
    xs = z.reshape(B, C, H * W).transpose(1, 0, 2).reshape(C, 1, S)
    lhs = jnp.stack([codes, jnp.ones_like(codes)], axis=1)
    base = (
        (jnp.arange(K, dtype=jnp.uint32) << jnp.uint32(1))
        | jnp.uint32(0x3F800000)
    ).reshape(1, K, 1)
    out_shape = [
        jax.ShapeDtypeStruct((C, 1, S), jnp.float32),
        jax.ShapeDtypeStruct((C, 1, S), jnp.float32),
        jax.ShapeDtypeStruct((C, 1, S), jnp.int32),
    ]
    grid = (C // _CPT, S // SBLK)
    soft, hard, idx = pl.pallas_call(
        _vq_tile,
        grid=grid,
        in_specs=[
            pl.BlockSpec((_CPT, 1, SBLK), lambda c, s: (c, 0, s)),
            pl.BlockSpec((_CPT, 2, K), lambda c, s: (c, 0, 0)),
            pl.BlockSpec((1, K, 1), lambda c, s: (0, 0, 0)),
        ],
        out_specs=[
            pl.BlockSpec((_CPT, 1, SBLK), lambda c, s: (c, 0, s)),
            pl.BlockSpec((_CPT, 1, SBLK), lambda c, s: (c, 0, s)),
            pl.BlockSpec((_CPT, 1, SBLK), lambda c, s: (c, 0, s)),
        ],
        out_shape=out_shape,
        compiler_params=pltpu.CompilerParams(
            dimension_semantics=("parallel", "parallel"),
        ),
        interpret=interpret,
    )(xs, lhs, base)

    def back(a):
        return a.reshape(C, B, H, W).transpose(1, 2, 3, 0)

    return back(soft), back(hard), back(idx)


def kernel(z, codes):
    return _run(z, codes)


# speedup vs baseline: 1.0248x; 1.0248x over previous
import functools

import jax
import jax.numpy as jnp
from jax.experimental import pallas as pl
from jax.experimental.pallas import tpu as pltpu

_NUM_CODES = 512
_LATENT = 64
_NEG_LOG2E = -1.4426950408889634


def _vq_tile(x_ref, lhs_ref, base_ref, soft_ref, hard_ref, idx_ref):
    x = x_ref[0, 0, :]
    w = lhs_ref[0][0, :].reshape(-1, 1)
    t = w - x[None, :]
    d = jnp.abs(t)
    dmin = jnp.min(d, axis=0, keepdims=True)
    e = jnp.exp2(d * jnp.float32(_NEG_LOG2E))
    nd = jax.lax.dot_general(
        lhs_ref[0], e, (((1,), (0,)), ((), ())),
        preferred_element_type=jnp.float32,
    )
    soft_ref[0, 0, :] = nd[0, :] / nd[1, :]
    sbit = jax.lax.shift_right_logical(
        jax.lax.bitcast_convert_type(t, jnp.uint32), jnp.uint32(31)
    )
    key = jax.lax.bitcast_convert_type(base_ref[0] | sbit, jnp.float32)
    packed = jax.lax.bitcast_convert_type(
        jnp.min(jnp.where(d == dmin, key, jnp.float32(2.0)), axis=0),
        jnp.uint32,
    )
    idx_ref[0, 0, :] = ((packed >> jnp.uint32(1)) & jnp.uint32(0x3FF)).astype(
        jnp.int32
    )
    sign = 1.0 - 2.0 * (packed & jnp.uint32(1)).astype(jnp.float32)
    hard_ref[0, 0, :] = x + sign * dmin[0, :]


@functools.partial(jax.jit, static_argnames=("interpret",))
def _run(z, codes, interpret=False):
    B, C, H, W = z.shape
    K = codes.shape[1]
    S = B * H * W
    SBLK = 2304
    xs = z.reshape(B, C, H * W).transpose(1, 0, 2).reshape(C, 1, S)
    lhs = jnp.stack([codes, jnp.ones_like(codes)], axis=1)
    base = (
        (jnp.arange(K, dtype=jnp.uint32) << jnp.uint32(1))
        | jnp.uint32(0x3F800000)
    ).reshape(1, K, 1)
    out_shape = [
        jax.ShapeDtypeStruct((C, 1, S), jnp.float32),
        jax.ShapeDtypeStruct((C, 1, S), jnp.float32),
        jax.ShapeDtypeStruct((C, 1, S), jnp.int32),
    ]
    grid = (C, S // SBLK)
    soft, hard, idx = pl.pallas_call(
        _vq_tile,
        grid=grid,
        in_specs=[
            pl.BlockSpec((1, 1, SBLK), lambda c, s: (c, 0, s)),
            pl.BlockSpec((1, 2, K), lambda c, s: (c, 0, 0)),
            pl.BlockSpec((1, K, 1), lambda c, s: (0, 0, 0)),
        ],
        out_specs=[
            pl.BlockSpec((1, 1, SBLK), lambda c, s: (c, 0, s)),
            pl.BlockSpec((1, 1, SBLK), lambda c, s: (c, 0, s)),
            pl.BlockSpec((1, 1, SBLK), lambda c, s: (c, 0, s)),
        ],
        out_shape=out_shape,
        compiler_params=pltpu.CompilerParams(
            dimension_semantics=("parallel", "parallel"),
        ),
        interpret=interpret,
    )(xs, lhs, base)

    def back(a):
        return a.reshape(C, B, H, W).transpose(1, 2, 3, 0)

    return back(soft), back(hard), back(idx)


def kernel(z, codes):
    return _run(z, codes)
